# baseline (speedup 1.0000x reference)
import functools

import jax
import jax.numpy as jnp
from jax import lax
from jax.experimental import pallas as pl
from jax.experimental.pallas import tpu as pltpu
from jax.experimental.pallas import tpu_sc as plsc

NUM_INPUTS = 160000
NUM_FEATURES = 256
NUM_OUTPUTS = 10000

NC = 2
NS = 16
FH = NUM_FEATURES // NC
CHUNK = 128
NCHUNKS = NUM_INPUTS // CHUNK
BASE_CHUNKS = NCHUNKS // NS
EXTRA_SUBS = NCHUNKS - NS * BASE_CHUNKS
NBUF = 3
NSTEPS = BASE_CHUNKS // NBUF
OCHUNK = 80
NOCHUNK = NUM_OUTPUTS // OCHUNK

_mesh = plsc.VectorSubcoreMesh(core_axis_name="c", subcore_axis_name="s")


@functools.partial(
    pl.kernel,
    out_type=jax.ShapeDtypeStruct((NUM_OUTPUTS, NUM_FEATURES), jnp.float32),
    mesh=_mesh,
    scratch_types=[
        [pltpu.VMEM((CHUNK,), jnp.int32) for _ in range(NBUF)],
        [pltpu.VMEM((CHUNK, FH), jnp.float32) for _ in range(NBUF)],
        [pltpu.SemaphoreType.DMA for _ in range(NBUF)],
        [pltpu.SemaphoreType.DMA for _ in range(NBUF)],
        pltpu.VMEM_SHARED((NUM_OUTPUTS, FH), jnp.float32),
    ],
)
def _sc_segment_sum(inp_hbm, idx_hbm, z_hbm, out_hbm,
                    idxs, rows, isems, rsems, acc):
    c = lax.axis_index("c")
    s = lax.axis_index("s")
    col0 = c * FH
    rbase = (s * BASE_CHUNKS + jnp.minimum(s, EXTRA_SUBS)) * CHUNK
    nchunk = jnp.where(s < EXTRA_SUBS, BASE_CHUNKS + 1, BASE_CHUNKS)
    nz = jnp.where(s < NOCHUNK - 7 * NS, 8, 7)

    def issue_loads(i, b):
        base = rbase + i * CHUNK
        pltpu.async_copy(idx_hbm.at[pl.ds(base, CHUNK)], idxs[b], isems[b])
        pltpu.async_copy(
            inp_hbm.at[pl.ds(base, CHUNK), pl.ds(col0, FH)], rows[b], rsems[b])

    def wait_loads(b):
        pltpu.make_async_copy(
            idx_hbm.at[pl.ds(0, CHUNK)], idxs[b], isems[b]).wait()
        pltpu.make_async_copy(
            inp_hbm.at[pl.ds(0, CHUNK), pl.ds(0, FH)], rows[b], rsems[b]).wait()

    for b in range(NBUF - 1):
        issue_loads(b, b)
    pltpu.sync_copy(z_hbm, rows[NBUF - 1].at[pl.ds(0, OCHUNK)])

    def zero_body(j, carry):
        r0 = (s + NS * j) * OCHUNK
        pltpu.sync_copy(rows[NBUF - 1].at[pl.ds(0, OCHUNK)],
                        acc.at[pl.ds(r0, OCHUNK)])
        return carry

    lax.fori_loop(0, nz, zero_body, 0)
    issue_loads(NBUF - 1, NBUF - 1)
    plsc.subcore_barrier()

    def step_body(step, carry):
        for b in range(NBUF):
            i = step * NBUF + b
            wait_loads(b)
            pltpu.sync_copy(rows[b], acc.at[idxs[b]], add=True)

            @pl.when(i + NBUF < nchunk)
            def _():
                issue_loads(i + NBUF, b)
        return carry

    lax.fori_loop(0, NSTEPS, step_body, 0)

    @pl.when(s < EXTRA_SUBS)
    def _():
        wait_loads(0)
        pltpu.sync_copy(rows[0], acc.at[idxs[0]], add=True)

    plsc.subcore_barrier()

    def out_body(j, carry):
        r0 = (s + NS * j) * OCHUNK
        pltpu.sync_copy(acc.at[pl.ds(r0, OCHUNK)],
                        rows[0].at[pl.ds(0, OCHUNK)])
        pltpu.sync_copy(rows[0].at[pl.ds(0, OCHUNK)],
                        out_hbm.at[pl.ds(r0, OCHUNK), pl.ds(col0, FH)])
        return carry

    lax.fori_loop(0, nz, out_body, 0)


def kernel(input, index, _):
    z = jnp.zeros((OCHUNK, FH), jnp.float32)
    out = _sc_segment_sum(input, index, z)
    return (input, index, out)
